# Initial kernel scaffold; baseline (speedup 1.0000x reference)
import jax
import jax.numpy as jnp
from jax.experimental import pallas as pl


def kernel(x, edge_index, W_in, b_in, W_out, b_out, att_l, att_r):
    raise NotImplementedError("write your pallas kernel here")


# speedup vs baseline: 6.2590x; 6.2590x over previous
import functools
import jax
import jax.numpy as jnp
from jax import lax
from jax.experimental import pallas as pl
from jax.experimental.pallas import tpu as pltpu
from jax.experimental.pallas import tpu_sc as plsc

N = 10000
E = 160000
H_DIM = 256
N_LAYERS = 4
EPS = 0.3

NC = 2
NS = 16
LANES = 16
HALF = 128

N_PAD = 10240
E_PAD = 163840
K = 128
ROWS_PER_TILE = N_PAD // NS
ROW_CHUNKS = ROWS_PER_TILE // K
EDGES_PER_TILE = E_PAD // NS
EDGE_CHUNKS = EDGES_PER_TILE // K
DEG_EDGES_PER_TILE = E_PAD // (NC * NS)
DEG_CHUNKS = DEG_EDGES_PER_TILE // K
TMP_W = 2048


def _tanh(z):
    e = jnp.exp(2.0 * z)
    return 1.0 - 2.0 / (e + 1.0)


def _rsqrt_newton(d):
    i = lax.bitcast_convert_type(d, jnp.int32)
    i = jnp.int32(0x5F3759DF) - lax.shift_right_logical(i, 1)
    y = lax.bitcast_convert_type(i, jnp.float32)
    for _ in range(3):
        y = y * (1.5 - 0.5 * d * y * y)
    return y


def _deg_body(dst_hbm, degp_out, dacc, dstbuf, onesbuf, zslice):
    c = lax.axis_index("c")
    s = lax.axis_index("s")
    wid = c * NS + s
    for i in range(ROWS_PER_TILE // LANES):
        zslice[pl.ds(i * LANES, LANES)] = jnp.zeros((LANES,), jnp.float32)
    pltpu.sync_copy(zslice, dacc.at[pl.ds(s * ROWS_PER_TILE, ROWS_PER_TILE)])
    for i in range(K // LANES):
        onesbuf[pl.ds(i * LANES, LANES)] = jnp.ones((LANES,), jnp.float32)
    plsc.subcore_barrier()

    def chunk(j, _):
        base = wid * DEG_EDGES_PER_TILE + j * K
        pltpu.sync_copy(dst_hbm.at[pl.ds(base, K)], dstbuf)
        pltpu.sync_copy(onesbuf, dacc.at[dstbuf], add=True)
        return 0

    lax.fori_loop(0, DEG_CHUNKS, chunk, 0)
    plsc.subcore_barrier()
    pltpu.sync_copy(dacc.at[pl.ds(s * ROWS_PER_TILE, ROWS_PER_TILE)], zslice)
    pltpu.sync_copy(zslice, degp_out.at[pl.ds(c * N_PAD + s * ROWS_PER_TILE,
                                              ROWS_PER_TILE)])


def _make_deg_kernel():
    mesh = plsc.VectorSubcoreMesh(core_axis_name="c", subcore_axis_name="s")
    return pl.kernel(
        _deg_body,
        out_type=jax.ShapeDtypeStruct((NC * N_PAD,), jnp.float32),
        mesh=mesh,
        compiler_params=pltpu.CompilerParams(needs_layout_passes=False),
        scratch_types=[
            pltpu.VMEM_SHARED((N_PAD,), jnp.float32),
            pltpu.VMEM((K,), jnp.int32),
            pltpu.VMEM((K,), jnp.float32),
            pltpu.VMEM((ROWS_PER_TILE,), jnp.float32),
        ],
    )


def _norm_body(degp_hbm, src_hbm, dst_hbm, norm_out,
               disbuf, tmpbuf, srcbuf, dstbuf, normbuf):
    c = lax.axis_index("c")
    s = lax.axis_index("s")
    wid = c * NS + s
    pltpu.sync_copy(degp_hbm.at[pl.ds(0, N_PAD)], disbuf)
    pltpu.sync_copy(degp_hbm.at[pl.ds(N_PAD, N_PAD)], tmpbuf)

    def combine(i, _):
        sl = pl.ds(i * LANES, LANES)
        d = jnp.maximum(disbuf[sl] + tmpbuf[sl], 1.0)
        disbuf[sl] = _rsqrt_newton(d)
        return 0

    lax.fori_loop(0, N_PAD // LANES, combine, 0)

    def chunk(j, _):
        base = wid * DEG_EDGES_PER_TILE + j * K
        pltpu.sync_copy(src_hbm.at[pl.ds(base, K)], srcbuf)
        pltpu.sync_copy(dst_hbm.at[pl.ds(base, K)], dstbuf)
        for g in range(K // LANES):
            sl = pl.ds(g * LANES, LANES)
            ds_ = plsc.load_gather(disbuf, [srcbuf[sl]])
            dd = plsc.load_gather(disbuf, [dstbuf[sl]])
            normbuf[sl] = ds_ * dd
        pltpu.sync_copy(normbuf, norm_out.at[pl.ds(base, K)])
        return 0

    lax.fori_loop(0, DEG_CHUNKS, chunk, 0)


def _make_norm_kernel():
    mesh = plsc.VectorSubcoreMesh(core_axis_name="c", subcore_axis_name="s")
    return pl.kernel(
        _norm_body,
        out_type=jax.ShapeDtypeStruct((E_PAD,), jnp.float32),
        mesh=mesh,
        compiler_params=pltpu.CompilerParams(needs_layout_passes=False),
        scratch_types=[
            pltpu.VMEM((N_PAD,), jnp.float32),
            pltpu.VMEM((N_PAD,), jnp.float32),
            pltpu.VMEM((K,), jnp.int32),
            pltpu.VMEM((K,), jnp.int32),
            pltpu.VMEM((K,), jnp.float32),
        ],
    )


def _layer_body(h_hbm, x0_hbm, alp_hbm, arp_hbm, norm_hbm, src_hbm, dst_hbm,
                attl_hbm, attr_hbm,
                hn_out, alpn_out, arpn_out,
                acc, albuf, arbuf, tmpbuf,
                srcbuf, dstbuf, gsrcbuf, normbuf, scalebuf,
                rows, attlbuf, attrbuf, alout, arout, sem):
    c = lax.axis_index("c")
    s = lax.axis_index("s")
    coff = c * N_PAD

    pltpu.sync_copy(alp_hbm.at[pl.ds(0, N_PAD)], albuf)
    pltpu.sync_copy(arp_hbm.at[pl.ds(0, N_PAD)], arbuf)

    def comb(ci, _):
        cb = ci * TMP_W
        pltpu.sync_copy(alp_hbm.at[pl.ds(N_PAD + cb, TMP_W)], tmpbuf)
        for i in range(TMP_W // LANES):
            sl = pl.ds(cb + i * LANES, LANES)
            albuf[sl] = albuf[sl] + tmpbuf[pl.ds(i * LANES, LANES)]
        pltpu.sync_copy(arp_hbm.at[pl.ds(N_PAD + cb, TMP_W)], tmpbuf)
        for i in range(TMP_W // LANES):
            sl = pl.ds(cb + i * LANES, LANES)
            arbuf[sl] = arbuf[sl] + tmpbuf[pl.ds(i * LANES, LANES)]
        return 0

    lax.fori_loop(0, N_PAD // TMP_W, comb, 0)
    pltpu.sync_copy(attl_hbm.at[pl.ds(c * HALF, HALF)], attlbuf)
    pltpu.sync_copy(attr_hbm.at[pl.ds(c * HALF, HALF)], attrbuf)

    for j in range(ROW_CHUNKS):
        rb = s * ROWS_PER_TILE + j * K
        pltpu.sync_copy(x0_hbm.at[pl.ds(coff + rb, K)], rows)

        def seed(i, _):
            for g in range(HALF // LANES):
                sl = pl.ds(g * LANES, LANES)
                rows[i, sl] = rows[i, sl] * EPS
            return 0

        lax.fori_loop(0, K, seed, 0)
        pltpu.sync_copy(rows, acc.at[pl.ds(rb, K)])
    plsc.subcore_barrier()

    def echunk(j, _):
        base = s * EDGES_PER_TILE + j * K
        pltpu.sync_copy(src_hbm.at[pl.ds(base, K)], srcbuf)
        pltpu.sync_copy(dst_hbm.at[pl.ds(base, K)], dstbuf)
        pltpu.sync_copy(norm_hbm.at[pl.ds(base, K)], normbuf)
        for g in range(K // LANES):
            sl = pl.ds(g * LANES, LANES)
            sv = srcbuf[sl]
            zl = plsc.load_gather(albuf, [sv])
            zr = plsc.load_gather(arbuf, [dstbuf[sl]])
            scalebuf[sl] = normbuf[sl] * _tanh(zl + zr)
            gsrcbuf[sl] = sv + coff
        pltpu.async_copy(h_hbm.at[gsrcbuf], rows, sem).wait()

        def scale_grp(gi, _):
            scv = scalebuf[pl.ds(gi * LANES, LANES)]
            for j in range(LANES):
                i = gi * LANES + j
                sc = jnp.broadcast_to(scv[j], (LANES,))
                for g in range(HALF // LANES):
                    sl = pl.ds(g * LANES, LANES)
                    rows[i, sl] = rows[i, sl] * sc
            return 0

        lax.fori_loop(0, K // LANES, scale_grp, 0)
        pltpu.sync_copy(rows, acc.at[dstbuf], add=True)
        return 0

    lax.fori_loop(0, EDGE_CHUNKS, echunk, 0)
    plsc.subcore_barrier()

    for j in range(ROW_CHUNKS):
        rb = s * ROWS_PER_TILE + j * K
        pltpu.sync_copy(acc.at[pl.ds(rb, K)], rows)

        def fingrp(gi, _):
            lane = lax.iota(jnp.int32, LANES)
            alvec = jnp.zeros((LANES,), jnp.float32)
            arvec = jnp.zeros((LANES,), jnp.float32)
            for jj in range(LANES):
                i = gi * LANES + jj
                alacc = jnp.zeros((LANES,), jnp.float32)
                aracc = jnp.zeros((LANES,), jnp.float32)
                for g in range(HALF // LANES):
                    sl = pl.ds(g * LANES, LANES)
                    hv = rows[i, sl]
                    alacc = alacc + hv * attlbuf[sl]
                    aracc = aracc + hv * attrbuf[sl]
                alvec = jnp.where(lane == jj, jnp.sum(alacc), alvec)
                arvec = jnp.where(lane == jj, jnp.sum(aracc), arvec)
            alout[pl.ds(j * K + gi * LANES, LANES)] = alvec
            arout[pl.ds(j * K + gi * LANES, LANES)] = arvec
            return 0

        lax.fori_loop(0, K // LANES, fingrp, 0)
        pltpu.sync_copy(rows, hn_out.at[pl.ds(coff + rb, K)])
    pltpu.sync_copy(alout, alpn_out.at[pl.ds(coff + s * ROWS_PER_TILE,
                                             ROWS_PER_TILE)])
    pltpu.sync_copy(arout, arpn_out.at[pl.ds(coff + s * ROWS_PER_TILE,
                                             ROWS_PER_TILE)])


def _make_layer_kernel():
    mesh = plsc.VectorSubcoreMesh(core_axis_name="c", subcore_axis_name="s")
    return pl.kernel(
        _layer_body,
        out_type=(
            jax.ShapeDtypeStruct((NC * N_PAD, HALF), jnp.float32),
            jax.ShapeDtypeStruct((NC * N_PAD,), jnp.float32),
            jax.ShapeDtypeStruct((NC * N_PAD,), jnp.float32),
        ),
        mesh=mesh,
        compiler_params=pltpu.CompilerParams(needs_layout_passes=False),
        scratch_types=[
            pltpu.VMEM_SHARED((N_PAD, HALF), jnp.float32),
            pltpu.VMEM((N_PAD,), jnp.float32),
            pltpu.VMEM((N_PAD,), jnp.float32),
            pltpu.VMEM((TMP_W,), jnp.float32),
            pltpu.VMEM((K,), jnp.int32),
            pltpu.VMEM((K,), jnp.int32),
            pltpu.VMEM((K,), jnp.int32),
            pltpu.VMEM((K,), jnp.float32),
            pltpu.VMEM((K,), jnp.float32),
            pltpu.VMEM((K, HALF), jnp.float32),
            pltpu.VMEM((HALF,), jnp.float32),
            pltpu.VMEM((HALF,), jnp.float32),
            pltpu.VMEM((ROWS_PER_TILE,), jnp.float32),
            pltpu.VMEM((ROWS_PER_TILE,), jnp.float32),
            pltpu.SemaphoreType.DMA,
        ],
    )


_BM = 512


def _mm_in_body(x_ref, w_ref, b_ref, att_ref, h_ref, alar_ref):
    h = jnp.maximum(jnp.dot(x_ref[...], w_ref[...],
                            preferred_element_type=jnp.float32)
                    + b_ref[0], 0.0)
    h_ref[0] = h
    alar_ref[0] = jnp.dot(h, att_ref[0], preferred_element_type=jnp.float32)


def _mm_in(x_pad, w_in, b_in2, attmat):
    grid = (N_PAD // _BM, 2)
    return pl.pallas_call(
        _mm_in_body,
        grid=grid,
        in_specs=[
            pl.BlockSpec((_BM, H_DIM), lambda i, j: (i, 0)),
            pl.BlockSpec((H_DIM, HALF), lambda i, j: (0, j)),
            pl.BlockSpec((1, 1, HALF), lambda i, j: (j, 0, 0)),
            pl.BlockSpec((1, HALF, HALF), lambda i, j: (j, 0, 0)),
        ],
        out_specs=[
            pl.BlockSpec((1, _BM, HALF), lambda i, j: (j, i, 0)),
            pl.BlockSpec((1, _BM, HALF), lambda i, j: (j, i, 0)),
        ],
        out_shape=[
            jax.ShapeDtypeStruct((NC, N_PAD, HALF), jnp.float32),
            jax.ShapeDtypeStruct((NC, N_PAD, HALF), jnp.float32),
        ],
    )(x_pad, w_in, b_in2, attmat)


def _mm_out_body(h_ref, w_ref, b_ref, o_ref):
    hcat = jnp.concatenate([h_ref[0], h_ref[1]], axis=1)
    wcat = jnp.concatenate([w_ref[0], w_ref[1]], axis=0)
    o_ref[...] = jnp.dot(hcat, wcat, preferred_element_type=jnp.float32) \
        + b_ref[0]


def _mm_out(h_halves, w_out3, b_out2):
    grid = (N_PAD // _BM, 2)
    return pl.pallas_call(
        _mm_out_body,
        grid=grid,
        in_specs=[
            pl.BlockSpec((NC, _BM, HALF), lambda i, j: (0, i, 0)),
            pl.BlockSpec((NC, HALF, HALF), lambda i, j: (0, 0, j)),
            pl.BlockSpec((1, 1, HALF), lambda i, j: (j, 0, 0)),
        ],
        out_specs=pl.BlockSpec((_BM, HALF), lambda i, j: (i, j)),
        out_shape=jax.ShapeDtypeStruct((N_PAD, H_DIM), jnp.float32),
    )(h_halves, w_out3, b_out2)


@jax.jit
def _run(x, edge_index, W_in, b_in, W_out, b_out, att_l, att_r):
    x_pad = jnp.zeros((N_PAD, H_DIM), jnp.float32).at[:N].set(x)
    src = edge_index[0].astype(jnp.int32)
    dst = edge_index[1].astype(jnp.int32)
    pad_idx = jnp.full((E_PAD - E,), N_PAD - 1, jnp.int32)
    src = jnp.concatenate([src, pad_idx])
    dst = jnp.concatenate([dst, pad_idx])

    b_in2 = b_in.reshape(2, 1, HALF)
    attmat = jnp.zeros((NC, HALF, HALF), jnp.float32)
    attmat = attmat.at[:, :, 0].set(att_l[0].reshape(NC, HALF))
    attmat = attmat.at[:, :, 1].set(att_r[0].reshape(NC, HALF))

    h2, alar = _mm_in(x_pad, W_in, b_in2, attmat)
    h = h2.reshape(NC * N_PAD, HALF)
    x0 = h
    alp = alar[:, :, 0].reshape(NC * N_PAD)
    arp = alar[:, :, 1].reshape(NC * N_PAD)

    degp = _make_deg_kernel()(dst)
    normv = _make_norm_kernel()(degp, src, dst)

    layer = _make_layer_kernel()
    zeros_att = jnp.zeros((H_DIM,), jnp.float32)
    for l in range(N_LAYERS):
        attl_n = att_l[l + 1] if l + 1 < N_LAYERS else zeros_att
        attr_n = att_r[l + 1] if l + 1 < N_LAYERS else zeros_att
        h, alp, arp = layer(h, x0, alp, arp, normv, src, dst, attl_n, attr_n)

    out = _mm_out(h.reshape(NC, N_PAD, HALF), W_out.reshape(NC, HALF, H_DIM),
                  b_out.reshape(2, 1, HALF))
    return out[:N]


def kernel(x, edge_index, W_in, b_in, W_out, b_out, att_l, att_r):
    return _run(x, edge_index, W_in, b_in, W_out, b_out, att_l, att_r)


# speedup vs baseline: 8.3736x; 1.3378x over previous
import jax
import jax.numpy as jnp
from jax import lax
from jax.experimental import pallas as pl
from jax.experimental.pallas import tpu as pltpu
from jax.experimental.pallas import tpu_sc as plsc

N = 10000
E = 160000
H_DIM = 256
N_LAYERS = 4
EPS = 0.3

NC = 2
NS = 16
LANES = 16
HALF = 128

N_PAD = 10240
E_PAD = 163840
K = 128
ROWS_PER_TILE = N_PAD // NS
ROW_CHUNKS = ROWS_PER_TILE // K
EDGES_PER_TILE = E_PAD // NS
EDGE_CHUNKS = EDGES_PER_TILE // K
DEG_EDGES_PER_TILE = E_PAD // (NC * NS)
DEG_CHUNKS = DEG_EDGES_PER_TILE // K
TMP_W = 2048
IB = 1024
IB_CHUNKS = IB // K
SC_BLOCKS = DEG_EDGES_PER_TILE // IB


def _tanh(z):
    e = jnp.exp(2.0 * z)
    return 1.0 - 2.0 / (e + 1.0)


def _rsqrt_newton(d):
    i = lax.bitcast_convert_type(d, jnp.int32)
    i = jnp.int32(0x5F3759DF) - lax.shift_right_logical(i, 1)
    y = lax.bitcast_convert_type(i, jnp.float32)
    for _ in range(3):
        y = y * (1.5 - 0.5 * d * y * y)
    return y


def _deg_body(dst_hbm, degp_out, dacc, dstbuf, onesbuf, zslice):
    c = lax.axis_index("c")
    s = lax.axis_index("s")
    wid = c * NS + s
    for i in range(ROWS_PER_TILE // LANES):
        zslice[pl.ds(i * LANES, LANES)] = jnp.zeros((LANES,), jnp.float32)
    pltpu.sync_copy(zslice, dacc.at[pl.ds(s * ROWS_PER_TILE, ROWS_PER_TILE)])
    for i in range(K // LANES):
        onesbuf[pl.ds(i * LANES, LANES)] = jnp.ones((LANES,), jnp.float32)
    plsc.subcore_barrier()

    def chunk(j, _):
        base = wid * DEG_EDGES_PER_TILE + j * K
        pltpu.sync_copy(dst_hbm.at[pl.ds(base, K)], dstbuf)
        pltpu.sync_copy(onesbuf, dacc.at[dstbuf], add=True)
        return 0

    lax.fori_loop(0, DEG_CHUNKS, chunk, 0)
    plsc.subcore_barrier()
    pltpu.sync_copy(dacc.at[pl.ds(s * ROWS_PER_TILE, ROWS_PER_TILE)], zslice)
    pltpu.sync_copy(zslice, degp_out.at[pl.ds(c * N_PAD + s * ROWS_PER_TILE,
                                              ROWS_PER_TILE)])


def _make_deg_kernel():
    mesh = plsc.VectorSubcoreMesh(core_axis_name="c", subcore_axis_name="s")
    return pl.kernel(
        _deg_body,
        out_type=jax.ShapeDtypeStruct((NC * N_PAD,), jnp.float32),
        mesh=mesh,
        compiler_params=pltpu.CompilerParams(needs_layout_passes=False),
        scratch_types=[
            pltpu.VMEM_SHARED((N_PAD,), jnp.float32),
            pltpu.VMEM((K,), jnp.int32),
            pltpu.VMEM((K,), jnp.float32),
            pltpu.VMEM((ROWS_PER_TILE,), jnp.float32),
        ],
    )


def _norm_body(degp_hbm, src_hbm, dst_hbm, norm_out,
               disbuf, tmpbuf, srcbuf, dstbuf, normbuf):
    c = lax.axis_index("c")
    s = lax.axis_index("s")
    wid = c * NS + s
    pltpu.sync_copy(degp_hbm.at[pl.ds(0, N_PAD)], disbuf)
    pltpu.sync_copy(degp_hbm.at[pl.ds(N_PAD, N_PAD)], tmpbuf)

    def combine(i, _):
        sl = pl.ds(i * LANES, LANES)
        d = jnp.maximum(disbuf[sl] + tmpbuf[sl], 1.0)
        disbuf[sl] = _rsqrt_newton(d)
        return 0

    lax.fori_loop(0, N_PAD // LANES, combine, 0)

    def chunk(j, _):
        base = wid * DEG_EDGES_PER_TILE + j * K
        pltpu.sync_copy(src_hbm.at[pl.ds(base, K)], srcbuf)
        pltpu.sync_copy(dst_hbm.at[pl.ds(base, K)], dstbuf)
        for g in range(K // LANES):
            sl = pl.ds(g * LANES, LANES)
            ds_ = plsc.load_gather(disbuf, [srcbuf[sl]])
            dd = plsc.load_gather(disbuf, [dstbuf[sl]])
            normbuf[sl] = ds_ * dd
        pltpu.sync_copy(normbuf, norm_out.at[pl.ds(base, K)])
        return 0

    lax.fori_loop(0, DEG_CHUNKS, chunk, 0)


def _make_norm_kernel():
    mesh = plsc.VectorSubcoreMesh(core_axis_name="c", subcore_axis_name="s")
    return pl.kernel(
        _norm_body,
        out_type=jax.ShapeDtypeStruct((E_PAD,), jnp.float32),
        mesh=mesh,
        compiler_params=pltpu.CompilerParams(needs_layout_passes=False),
        scratch_types=[
            pltpu.VMEM((N_PAD,), jnp.float32),
            pltpu.VMEM((N_PAD,), jnp.float32),
            pltpu.VMEM((K,), jnp.int32),
            pltpu.VMEM((K,), jnp.int32),
            pltpu.VMEM((K,), jnp.float32),
        ],
    )


def _scale_body(alp_hbm, arp_hbm, norm_hbm, src_hbm, dst_hbm, scale_out,
                albuf, arbuf, tmpbuf, sstage, dstage, nstage, ostage):
    c = lax.axis_index("c")
    s = lax.axis_index("s")
    wid = c * NS + s
    pltpu.sync_copy(alp_hbm.at[pl.ds(0, N_PAD)], albuf)
    pltpu.sync_copy(arp_hbm.at[pl.ds(0, N_PAD)], arbuf)

    def comb(ci, _):
        cb = ci * TMP_W
        pltpu.sync_copy(alp_hbm.at[pl.ds(N_PAD + cb, TMP_W)], tmpbuf)
        for i in range(TMP_W // LANES):
            sl = pl.ds(cb + i * LANES, LANES)
            albuf[sl] = albuf[sl] + tmpbuf[pl.ds(i * LANES, LANES)]
        pltpu.sync_copy(arp_hbm.at[pl.ds(N_PAD + cb, TMP_W)], tmpbuf)
        for i in range(TMP_W // LANES):
            sl = pl.ds(cb + i * LANES, LANES)
            arbuf[sl] = arbuf[sl] + tmpbuf[pl.ds(i * LANES, LANES)]
        return 0

    lax.fori_loop(0, N_PAD // TMP_W, comb, 0)

    def block(t, _):
        sb = wid * DEG_EDGES_PER_TILE + t * IB
        pltpu.sync_copy(src_hbm.at[pl.ds(sb, IB)], sstage)
        pltpu.sync_copy(dst_hbm.at[pl.ds(sb, IB)], dstage)
        pltpu.sync_copy(norm_hbm.at[pl.ds(sb, IB)], nstage)

        def grp(g, _):
            sl = pl.ds(g * LANES, LANES)
            zl = plsc.load_gather(albuf, [sstage[sl]])
            zr = plsc.load_gather(arbuf, [dstage[sl]])
            ostage[sl] = nstage[sl] * _tanh(zl + zr)
            return 0

        lax.fori_loop(0, IB // LANES, grp, 0)
        pltpu.sync_copy(ostage, scale_out.at[pl.ds(sb, IB)])
        return 0

    lax.fori_loop(0, SC_BLOCKS, block, 0)


def _make_scale_kernel():
    mesh = plsc.VectorSubcoreMesh(core_axis_name="c", subcore_axis_name="s")
    return pl.kernel(
        _scale_body,
        out_type=jax.ShapeDtypeStruct((E_PAD,), jnp.float32),
        mesh=mesh,
        compiler_params=pltpu.CompilerParams(needs_layout_passes=False),
        scratch_types=[
            pltpu.VMEM((N_PAD,), jnp.float32),
            pltpu.VMEM((N_PAD,), jnp.float32),
            pltpu.VMEM((TMP_W,), jnp.float32),
            pltpu.VMEM((IB,), jnp.int32),
            pltpu.VMEM((IB,), jnp.int32),
            pltpu.VMEM((IB,), jnp.float32),
            pltpu.VMEM((IB,), jnp.float32),
        ],
    )


def _layer_body(h_hbm, x0_hbm, scale_hbm, src_hbm, dst_hbm,
                attl_hbm, attr_hbm,
                hn_out, alpn_out, arpn_out,
                acc, sstage, dstage, cstage,
                gsrc0, gsrc1, dstb0, dstb1, scb0, scb1, rows0, rows1,
                attlbuf, attrbuf, alout, arout,
                gsem0, gsem1, ssem0, ssem1):
    c = lax.axis_index("c")
    s = lax.axis_index("s")
    coff = c * N_PAD
    gsrc = (gsrc0, gsrc1)
    dstb = (dstb0, dstb1)
    scb = (scb0, scb1)
    rows = (rows0, rows1)
    gsem = (gsem0, gsem1)
    ssem = (ssem0, ssem1)

    pltpu.sync_copy(attl_hbm.at[pl.ds(c * HALF, HALF)], attlbuf)
    pltpu.sync_copy(attr_hbm.at[pl.ds(c * HALF, HALF)], attrbuf)
    for j in range(ROW_CHUNKS):
        rb = s * ROWS_PER_TILE + j * K
        pltpu.sync_copy(x0_hbm.at[pl.ds(coff + rb, K)], rows0)

        def seed(i, _):
            for g in range(HALF // LANES):
                sl = pl.ds(g * LANES, LANES)
                rows0[i, sl] = rows0[i, sl] * EPS
            return 0

        lax.fori_loop(0, K, seed, 0)
        pltpu.sync_copy(rows0, acc.at[pl.ds(rb, K)])
    plsc.subcore_barrier()

    ebase = s * EDGES_PER_TILE

    def stage_block(t):
        sb = ebase + t * IB
        pltpu.sync_copy(src_hbm.at[pl.ds(sb, IB)], sstage)
        pltpu.sync_copy(dst_hbm.at[pl.ds(sb, IB)], dstage)
        pltpu.sync_copy(scale_hbm.at[pl.ds(sb, IB)], cstage)

    def prep_and_fire(jl, b):
        ob = jl * K
        for g in range(K // LANES):
            sl = pl.ds(ob + g * LANES, LANES)
            ll = pl.ds(g * LANES, LANES)
            gsrc[b][ll] = sstage[sl] + coff
            dstb[b][ll] = dstage[sl]
            scb[b][ll] = cstage[sl]
        pltpu.async_copy(h_hbm.at[gsrc[b]], rows[b], gsem[b])

    def wait_gather(b):
        pltpu.make_async_copy(h_hbm.at[gsrc[b]], rows[b], gsem[b]).wait()

    def fire_scatter(b):
        pltpu.async_copy(rows[b], acc.at[dstb[b]], ssem[b], add=True)

    def wait_scatter(b):
        pltpu.make_async_copy(rows[b], acc.at[dstb[b]], ssem[b]).wait()

    def scale_rows(b):
        def sg(gi, _):
            scv = scb[b][pl.ds(gi * LANES, LANES)]
            for jj in range(LANES):
                i = gi * LANES + jj
                sc = jnp.broadcast_to(scv[jj], (LANES,))
                for g in range(HALF // LANES):
                    sl = pl.ds(g * LANES, LANES)
                    rows[b][i, sl] = rows[b][i, sl] * sc
            return 0

        lax.fori_loop(0, K // LANES, sg, 0)

    stage_block(0)
    prep_and_fire(0, 0)

    def pair(jj, _):
        for b in range(2):
            j = 2 * jj + b
            nb = 1 - b
            wait_gather(b)
            pl.when(j >= 1)(lambda: wait_scatter(nb))
            jn = j + 1

            def prep_next():
                jl = lax.rem(jn, IB_CHUNKS)
                pl.when(jl == 0)(
                    lambda: stage_block(lax.div(jn, IB_CHUNKS)))
                prep_and_fire(jl, nb)

            pl.when(jn < EDGE_CHUNKS)(prep_next)
            scale_rows(b)
            fire_scatter(b)
        return 0

    lax.fori_loop(0, EDGE_CHUNKS // 2, pair, 0)
    wait_scatter(1)
    plsc.subcore_barrier()

    for j in range(ROW_CHUNKS):
        rb = s * ROWS_PER_TILE + j * K
        pltpu.sync_copy(acc.at[pl.ds(rb, K)], rows0)

        def fingrp(gi, _):
            lane = lax.iota(jnp.int32, LANES)
            alvec = jnp.zeros((LANES,), jnp.float32)
            arvec = jnp.zeros((LANES,), jnp.float32)
            for jj in range(LANES):
                i = gi * LANES + jj
                alacc = jnp.zeros((LANES,), jnp.float32)
                aracc = jnp.zeros((LANES,), jnp.float32)
                for g in range(HALF // LANES):
                    sl = pl.ds(g * LANES, LANES)
                    hv = rows0[i, sl]
                    alacc = alacc + hv * attlbuf[sl]
                    aracc = aracc + hv * attrbuf[sl]
                alvec = jnp.where(lane == jj, jnp.sum(alacc), alvec)
                arvec = jnp.where(lane == jj, jnp.sum(aracc), arvec)
            alout[pl.ds(j * K + gi * LANES, LANES)] = alvec
            arout[pl.ds(j * K + gi * LANES, LANES)] = arvec
            return 0

        lax.fori_loop(0, K // LANES, fingrp, 0)
        pltpu.sync_copy(rows0, hn_out.at[pl.ds(coff + rb, K)])
    pltpu.sync_copy(alout, alpn_out.at[pl.ds(coff + s * ROWS_PER_TILE,
                                             ROWS_PER_TILE)])
    pltpu.sync_copy(arout, arpn_out.at[pl.ds(coff + s * ROWS_PER_TILE,
                                             ROWS_PER_TILE)])


def _make_layer_kernel():
    mesh = plsc.VectorSubcoreMesh(core_axis_name="c", subcore_axis_name="s")
    return pl.kernel(
        _layer_body,
        out_type=(
            jax.ShapeDtypeStruct((NC * N_PAD, HALF), jnp.float32),
            jax.ShapeDtypeStruct((NC * N_PAD,), jnp.float32),
            jax.ShapeDtypeStruct((NC * N_PAD,), jnp.float32),
        ),
        mesh=mesh,
        compiler_params=pltpu.CompilerParams(needs_layout_passes=False),
        scratch_types=[
            pltpu.VMEM_SHARED((N_PAD, HALF), jnp.float32),
            pltpu.VMEM((IB,), jnp.int32),
            pltpu.VMEM((IB,), jnp.int32),
            pltpu.VMEM((IB,), jnp.float32),
            pltpu.VMEM((K,), jnp.int32),
            pltpu.VMEM((K,), jnp.int32),
            pltpu.VMEM((K,), jnp.int32),
            pltpu.VMEM((K,), jnp.int32),
            pltpu.VMEM((K,), jnp.float32),
            pltpu.VMEM((K,), jnp.float32),
            pltpu.VMEM((K, HALF), jnp.float32),
            pltpu.VMEM((K, HALF), jnp.float32),
            pltpu.VMEM((HALF,), jnp.float32),
            pltpu.VMEM((HALF,), jnp.float32),
            pltpu.VMEM((ROWS_PER_TILE,), jnp.float32),
            pltpu.VMEM((ROWS_PER_TILE,), jnp.float32),
            pltpu.SemaphoreType.DMA,
            pltpu.SemaphoreType.DMA,
            pltpu.SemaphoreType.DMA,
            pltpu.SemaphoreType.DMA,
        ],
    )


_BM = 512


def _mm_in_body(x_ref, w_ref, b_ref, att_ref, h_ref, alar_ref):
    h = jnp.maximum(jnp.dot(x_ref[...], w_ref[...],
                            preferred_element_type=jnp.float32)
                    + b_ref[0], 0.0)
    h_ref[0] = h
    alar_ref[0] = jnp.dot(h, att_ref[0], preferred_element_type=jnp.float32)


def _mm_in(x_pad, w_in, b_in2, attmat):
    grid = (N_PAD // _BM, 2)
    return pl.pallas_call(
        _mm_in_body,
        grid=grid,
        in_specs=[
            pl.BlockSpec((_BM, H_DIM), lambda i, j: (i, 0)),
            pl.BlockSpec((H_DIM, HALF), lambda i, j: (0, j)),
            pl.BlockSpec((1, 1, HALF), lambda i, j: (j, 0, 0)),
            pl.BlockSpec((1, HALF, HALF), lambda i, j: (j, 0, 0)),
        ],
        out_specs=[
            pl.BlockSpec((1, _BM, HALF), lambda i, j: (j, i, 0)),
            pl.BlockSpec((1, _BM, HALF), lambda i, j: (j, i, 0)),
        ],
        out_shape=[
            jax.ShapeDtypeStruct((NC, N_PAD, HALF), jnp.float32),
            jax.ShapeDtypeStruct((NC, N_PAD, HALF), jnp.float32),
        ],
    )(x_pad, w_in, b_in2, attmat)


def _mm_out_body(h_ref, w_ref, b_ref, o_ref):
    hcat = jnp.concatenate([h_ref[0], h_ref[1]], axis=1)
    wcat = jnp.concatenate([w_ref[0], w_ref[1]], axis=0)
    o_ref[...] = jnp.dot(hcat, wcat, preferred_element_type=jnp.float32) \
        + b_ref[0]


def _mm_out(h_halves, w_out3, b_out2):
    grid = (N_PAD // _BM, 2)
    return pl.pallas_call(
        _mm_out_body,
        grid=grid,
        in_specs=[
            pl.BlockSpec((NC, _BM, HALF), lambda i, j: (0, i, 0)),
            pl.BlockSpec((NC, HALF, HALF), lambda i, j: (0, 0, j)),
            pl.BlockSpec((1, 1, HALF), lambda i, j: (j, 0, 0)),
        ],
        out_specs=pl.BlockSpec((_BM, HALF), lambda i, j: (i, j)),
        out_shape=jax.ShapeDtypeStruct((N_PAD, H_DIM), jnp.float32),
    )(h_halves, w_out3, b_out2)


@jax.jit
def _run(x, edge_index, W_in, b_in, W_out, b_out, att_l, att_r):
    x_pad = jnp.zeros((N_PAD, H_DIM), jnp.float32).at[:N].set(x)
    src = edge_index[0].astype(jnp.int32)
    dst = edge_index[1].astype(jnp.int32)
    pad_idx = jnp.full((E_PAD - E,), N_PAD - 1, jnp.int32)
    src = jnp.concatenate([src, pad_idx])
    dst = jnp.concatenate([dst, pad_idx])

    b_in2 = b_in.reshape(2, 1, HALF)
    attmat = jnp.zeros((NC, HALF, HALF), jnp.float32)
    attmat = attmat.at[:, :, 0].set(att_l[0].reshape(NC, HALF))
    attmat = attmat.at[:, :, 1].set(att_r[0].reshape(NC, HALF))

    h2, alar = _mm_in(x_pad, W_in, b_in2, attmat)
    h = h2.reshape(NC * N_PAD, HALF)
    x0 = h
    alp = alar[:, :, 0].reshape(NC * N_PAD)
    arp = alar[:, :, 1].reshape(NC * N_PAD)

    degp = _make_deg_kernel()(dst)
    normv = _make_norm_kernel()(degp, src, dst)

    scale_k = _make_scale_kernel()
    layer = _make_layer_kernel()
    zeros_att = jnp.zeros((H_DIM,), jnp.float32)
    for l in range(N_LAYERS):
        attl_n = att_l[l + 1] if l + 1 < N_LAYERS else zeros_att
        attr_n = att_r[l + 1] if l + 1 < N_LAYERS else zeros_att
        scalev = scale_k(alp, arp, normv, src, dst)
        h, alp, arp = layer(h, x0, scalev, src, dst, attl_n, attr_n)

    out = _mm_out(h.reshape(NC, N_PAD, HALF), W_out.reshape(NC, HALF, H_DIM),
                  b_out.reshape(2, 1, HALF))
    return out[:N]


def kernel(x, edge_index, W_in, b_in, W_out, b_out, att_l, att_r):
    return _run(x, edge_index, W_in, b_in, W_out, b_out, att_l, att_r)
